# baseline (speedup 1.0000x reference)
import functools

import jax
import jax.numpy as jnp
import numpy as np
from jax import lax
from jax.experimental import pallas as pl
from jax.experimental import pallas as pl
from jax.experimental.pallas import tpu as pltpu
from jax.experimental.pallas import tpu_sc as plsc

_FIELD_DIMS = [1000] * 26
_EMBED_DIM = 128
_BATCH = 16384
_NUM_FIELDS = 26
_PAD_F = 32
_OFF = np.array((0, *np.cumsum(_FIELD_DIMS)[:-1]), dtype=np.int32)

_NC = 2
_NS = 16
_NW = _NC * _NS
_TOTAL = _BATCH * _PAD_F
_ROWS_W = _TOTAL // _NW
_BLK = 128
_NBLK = _ROWS_W // _BLK
_XROWS = _TOTAL // _BLK
_XROWS_W = _XROWS // _NW


def _body(x_hbm, off_hbm, w_hbm, out_hbm, x_v, off_v, rows0, rows1, rows2,
          rows3, gsem, wsem):
    c = lax.axis_index("c")
    s = lax.axis_index("s")
    wid = s * _NC + c
    xrow0 = wid * _XROWS_W
    out0 = wid * _ROWS_W
    rows = (rows0, rows1, rows2, rows3)

    pltpu.sync_copy(x_hbm.at[pl.ds(xrow0, _XROWS_W)], x_v)
    pltpu.sync_copy(off_hbm, off_v)

    def compute_idx(r, carry):
        for c8 in range(8):
            sl = pl.ds(c8 * 16, 16)
            x_v[r, sl] = x_v[r, sl] + off_v[pl.ds((c8 % 2) * 16, 16)]
        return carry

    lax.fori_loop(0, _XROWS_W, compute_idx, 0)

    def out_at(g):
        return out_hbm.at[pl.ds(out0 + g * _BLK, _BLK)]

    pltpu.async_copy(w_hbm.at[x_v.at[0]], rows0, gsem)
    pltpu.async_copy(w_hbm.at[x_v.at[1]], rows1, gsem)

    def step(i, carry):
        for j in range(4):
            g = 4 * i + j
            pltpu.make_async_copy(w_hbm.at[x_v.at[g]], rows[j], gsem).wait()
            pltpu.async_copy(rows[j], out_at(g), wsem)
            drain = pltpu.make_async_copy(rows[j], out_at(g), wsem)
            if j < 2:

                @pl.when(i >= 1)
                def _():
                    drain.wait()

                pltpu.async_copy(w_hbm.at[x_v.at[g + 2]], rows[j + 2], gsem)
            else:
                drain.wait()

                @pl.when(i < _NBLK // 4 - 1)
                def _():
                    pltpu.async_copy(
                        w_hbm.at[x_v.at[g + 2]], rows[(j + 2) % 4], gsem)

        return carry

    lax.fori_loop(0, _NBLK // 4, step, 0)

    pltpu.make_async_copy(rows2, out_at(_NBLK - 2), wsem).wait()
    pltpu.make_async_copy(rows3, out_at(_NBLK - 1), wsem).wait()


@jax.jit
def kernel(x, weight):
    xp = jnp.pad(x, ((0, 0), (0, _PAD_F - _NUM_FIELDS)))
    x2d = xp.reshape(_XROWS, _BLK)
    off = jnp.concatenate(
        [jnp.asarray(_OFF), jnp.zeros(_PAD_F - _NUM_FIELDS, jnp.int32)])
    mesh = plsc.VectorSubcoreMesh(core_axis_name="c", subcore_axis_name="s")
    out = pl.kernel(
        _body,
        out_type=jax.ShapeDtypeStruct((_TOTAL, _EMBED_DIM), jnp.float32),
        mesh=mesh,
        scratch_types=[
            pltpu.VMEM((_XROWS_W, _BLK), jnp.int32),
            pltpu.VMEM((_PAD_F,), jnp.int32),
            pltpu.VMEM((_BLK, _EMBED_DIM), jnp.float32),
            pltpu.VMEM((_BLK, _EMBED_DIM), jnp.float32),
            pltpu.VMEM((_BLK, _EMBED_DIM), jnp.float32),
            pltpu.VMEM((_BLK, _EMBED_DIM), jnp.float32),
            pltpu.SemaphoreType.DMA,
            pltpu.SemaphoreType.DMA,
        ],
    )(x2d, off, weight)
    out = out.reshape(_BATCH, _PAD_F, _EMBED_DIM)
    return out[:, :_NUM_FIELDS, :]


# speedup vs baseline: 12.4321x; 12.4321x over previous
import functools

import jax
import jax.numpy as jnp
import numpy as np
from jax import lax
from jax.experimental import pallas as pl
from jax.experimental.pallas import tpu as pltpu
from jax.experimental.pallas import tpu_sc as plsc

_FIELD_DIMS = [1000] * 26
_EMBED_DIM = 128
_BATCH = 16384
_NUM_FIELDS = 26
_OFF = np.array((0, *np.cumsum(_FIELD_DIMS)[:-1]), dtype=np.int32)

_NC = 2
_NS = 16
_NW = _NC * _NS
_TOTAL = _BATCH * _NUM_FIELDS
_IDX_W = _TOTAL // _NW
_BATCH_W = _BATCH // _NW
_BPB = 4
_BLK = _BPB * _NUM_FIELDS
_NBLK = _BATCH_W // _BPB
_PAT = 208
_SL_PER_PAT = _PAT // 16


def _body(x_hbm, off_hbm, w_hbm, out_hbm, x_v, off_v, rows0, rows1, rows2,
          rows3, gsem, wsem):
    c = lax.axis_index("c")
    s = lax.axis_index("s")
    wid = s * _NC + c
    b0 = wid * _BATCH_W
    rows = (rows0, rows1, rows2, rows3)

    pltpu.sync_copy(x_hbm.at[pl.ds(wid * _IDX_W, _IDX_W)], x_v)
    pltpu.sync_copy(off_hbm, off_v)

    def compute_idx(j, carry):
        for t in range(_SL_PER_PAT):
            x_v[pl.ds(j * _PAT + t * 16, 16)] = (
                x_v[pl.ds(j * _PAT + t * 16, 16)] + off_v[pl.ds(t * 16, 16)])
        return carry

    lax.fori_loop(0, _IDX_W // _PAT, compute_idx, 0)

    def gather(g, buf):
        pltpu.async_copy(w_hbm.at[x_v.at[pl.ds(g * _BLK, _BLK)]], buf, gsem)

    def write(g, buf):
        for i in range(_BPB):
            pltpu.async_copy(
                buf.at[pl.ds(i * _NUM_FIELDS, _NUM_FIELDS)],
                out_hbm.at[b0 + g * _BPB + i], wsem)

    def drain_write(g, buf):
        for i in range(_BPB):
            pltpu.make_async_copy(
                buf.at[pl.ds(i * _NUM_FIELDS, _NUM_FIELDS)],
                out_hbm.at[b0 + g * _BPB + i], wsem).wait()

    gather(0, rows0)
    gather(1, rows1)

    def step(i, carry):
        for j in range(4):
            g = 4 * i + j
            pltpu.make_async_copy(
                w_hbm.at[x_v.at[pl.ds(g * _BLK, _BLK)]], rows[j], gsem).wait()
            write(g, rows[j])
            if j < 2:

                @pl.when(i >= 1)
                def _():
                    drain_write(g, rows[j])

                gather(g + 2, rows[j + 2])
            else:
                drain_write(g, rows[j])

                @pl.when(i < _NBLK // 4 - 1)
                def _():
                    gather(g + 2, rows[(j + 2) % 4])

        return carry

    lax.fori_loop(0, _NBLK // 4, step, 0)

    drain_write(_NBLK - 2, rows2)
    drain_write(_NBLK - 1, rows3)


@jax.jit
def kernel(x, weight):
    xf = x.reshape(_TOTAL)
    off = jnp.tile(jnp.asarray(_OFF), _PAT // _NUM_FIELDS)
    mesh = plsc.VectorSubcoreMesh(core_axis_name="c", subcore_axis_name="s")
    return pl.kernel(
        _body,
        out_type=jax.ShapeDtypeStruct((_BATCH, _NUM_FIELDS, _EMBED_DIM),
                                      jnp.float32),
        mesh=mesh,
        scratch_types=[
            pltpu.VMEM((_IDX_W,), jnp.int32),
            pltpu.VMEM((_PAT,), jnp.int32),
            pltpu.VMEM((_BLK, _EMBED_DIM), jnp.float32),
            pltpu.VMEM((_BLK, _EMBED_DIM), jnp.float32),
            pltpu.VMEM((_BLK, _EMBED_DIM), jnp.float32),
            pltpu.VMEM((_BLK, _EMBED_DIM), jnp.float32),
            pltpu.SemaphoreType.DMA,
            pltpu.SemaphoreType.DMA,
        ],
    )(xf, off, weight)


# speedup vs baseline: 22.0743x; 1.7756x over previous
import functools

import jax
import jax.numpy as jnp
import numpy as np
from jax import lax
from jax.experimental import pallas as pl
from jax.experimental.pallas import tpu as pltpu
from jax.experimental.pallas import tpu_sc as plsc

_EMBED_DIM = 128
_BATCH = 16384
_NUM_FIELDS = 26
_FIELD_DIM = 1000

_NC = 2
_NS = 16
_NW = _NC * _NS
_TOTAL = _BATCH * _NUM_FIELDS
_BATCH_W = _BATCH // _NW
_BLK = 128
_BPF = _BATCH_W // _BLK
_NBLK = _NUM_FIELDS * _BPF


def _body(x_hbm, w_hbm, out_hbm, idx_v, rows0, rows1, rows2, rows3,
          gsem, wsem):
    c = lax.axis_index("c")
    s = lax.axis_index("s")
    wid = s * _NC + c
    b0 = wid * _BATCH_W
    rows = (rows0, rows1, rows2, rows3)

    pltpu.sync_copy(x_hbm.at[:, pl.ds(wid * _BPF, _BPF), :], idx_v)

    def compute_idx(k, carry):
        for f in range(_NUM_FIELDS):
            for j in range(_BPF):
                sl = pl.ds(k * 16, 16)
                idx_v[f, j, sl] = idx_v[f, j, sl] + f * _FIELD_DIM
        return carry

    lax.fori_loop(0, _BLK // 16, compute_idx, 0)

    def gather(f, j, buf):
        pltpu.async_copy(w_hbm.at[idx_v.at[f, j]], buf, gsem)

    def out_at(f, j):
        return out_hbm.at[pl.ds(f * _BATCH + b0 + j * _BLK, _BLK)]

    gather(0, 0, rows0)
    gather(0, 1, rows1)

    def step(f, carry):
        for j in range(4):
            pltpu.make_async_copy(
                w_hbm.at[idx_v.at[f, j]], rows[j], gsem).wait()
            pltpu.async_copy(rows[j], out_at(f, j), wsem)
            drain = pltpu.make_async_copy(rows[j], out_at(f, j), wsem)
            if j < 2:

                @pl.when(f >= 1)
                def _():
                    drain.wait()

                gather(f, j + 2, rows[j + 2])
            else:
                drain.wait()

                @pl.when(f < _NUM_FIELDS - 1)
                def _():
                    gather(f + 1, j - 2, rows[(j + 2) % 4])

        return carry

    lax.fori_loop(0, _NUM_FIELDS, step, 0)

    pltpu.make_async_copy(rows2, out_at(_NUM_FIELDS - 1, 2), wsem).wait()
    pltpu.make_async_copy(rows3, out_at(_NUM_FIELDS - 1, 3), wsem).wait()


@jax.jit
def kernel(x, weight):
    xt = x.T.reshape(_NUM_FIELDS, _BATCH // _BLK, _BLK)
    mesh = plsc.VectorSubcoreMesh(core_axis_name="c", subcore_axis_name="s")
    out = pl.kernel(
        _body,
        out_type=jax.ShapeDtypeStruct((_TOTAL, _EMBED_DIM), jnp.float32),
        mesh=mesh,
        scratch_types=[
            pltpu.VMEM((_NUM_FIELDS, _BPF, _BLK), jnp.int32),
            pltpu.VMEM((_BLK, _EMBED_DIM), jnp.float32),
            pltpu.VMEM((_BLK, _EMBED_DIM), jnp.float32),
            pltpu.VMEM((_BLK, _EMBED_DIM), jnp.float32),
            pltpu.VMEM((_BLK, _EMBED_DIM), jnp.float32),
            pltpu.SemaphoreType.DMA,
            pltpu.SemaphoreType.DMA,
        ],
    )(xt, weight)
    return out.reshape(_NUM_FIELDS, _BATCH, _EMBED_DIM).transpose(1, 0, 2)


# speedup vs baseline: 26.3513x; 1.1938x over previous
import functools

import jax
import jax.numpy as jnp
import numpy as np
from jax import lax
from jax.experimental import pallas as pl
from jax.experimental.pallas import tpu as pltpu
from jax.experimental.pallas import tpu_sc as plsc

_EMBED_DIM = 128
_BATCH = 16384
_NUM_FIELDS = 26
_FIELD_DIM = 1000

_NC = 2
_NS = 16
_NW = _NC * _NS
_TOTAL = _BATCH * _NUM_FIELDS
_BATCH_W = _BATCH // _NW
_BLK = 128
_BPF = _BATCH_W // _BLK
_NBLK = _NUM_FIELDS * _BPF


def _body(x_hbm, w_hbm, out_hbm, idx_v, rows0, rows1, rows2, rows3,
          gsem, wsem):
    c = lax.axis_index("c")
    s = lax.axis_index("s")
    wid = s * _NC + c
    b0 = wid * _BATCH_W
    rows = (rows0, rows1, rows2, rows3)

    pltpu.sync_copy(x_hbm.at[:, pl.ds(wid * _BPF, _BPF), :], idx_v)

    def compute_idx(k, carry):
        for f in range(_NUM_FIELDS):
            for j in range(_BPF):
                sl = pl.ds(k * 16, 16)
                idx_v[f, j, sl] = idx_v[f, j, sl] + f * _FIELD_DIM
        return carry

    lax.fori_loop(0, _BLK // 16, compute_idx, 0)

    def gather(f, j, buf):
        pltpu.async_copy(w_hbm.at[idx_v.at[f, j]], buf, gsem)

    def out_at(f, j):
        return out_hbm.at[pl.ds(f * _BATCH + b0 + j * _BLK, _BLK)]

    def fld(i):
        return lax.rem(i + wid, _NUM_FIELDS)

    gather(fld(0), 0, rows0)
    gather(fld(0), 1, rows1)

    def step(i, carry):
        f = fld(i)
        for j in range(4):
            pltpu.make_async_copy(
                w_hbm.at[idx_v.at[f, j]], rows[j], gsem).wait()
            pltpu.async_copy(rows[j], out_at(f, j), wsem)
            drain = pltpu.make_async_copy(rows[j], out_at(f, j), wsem)
            if j < 2:

                @pl.when(i >= 1)
                def _():
                    drain.wait()

                gather(f, j + 2, rows[j + 2])
            else:
                drain.wait()

                @pl.when(i < _NUM_FIELDS - 1)
                def _():
                    gather(fld(i + 1), j - 2, rows[(j + 2) % 4])

        return carry

    lax.fori_loop(0, _NUM_FIELDS, step, 0)

    lastf = fld(_NUM_FIELDS - 1)
    pltpu.make_async_copy(rows2, out_at(lastf, 2), wsem).wait()
    pltpu.make_async_copy(rows3, out_at(lastf, 3), wsem).wait()


@jax.jit
def kernel(x, weight):
    xt = x.T.reshape(_NUM_FIELDS, _BATCH // _BLK, _BLK)
    mesh = plsc.VectorSubcoreMesh(core_axis_name="c", subcore_axis_name="s")
    out = pl.kernel(
        _body,
        out_type=jax.ShapeDtypeStruct((_TOTAL, _EMBED_DIM), jnp.float32),
        mesh=mesh,
        scratch_types=[
            pltpu.VMEM((_NUM_FIELDS, _BPF, _BLK), jnp.int32),
            pltpu.VMEM((_BLK, _EMBED_DIM), jnp.float32),
            pltpu.VMEM((_BLK, _EMBED_DIM), jnp.float32),
            pltpu.VMEM((_BLK, _EMBED_DIM), jnp.float32),
            pltpu.VMEM((_BLK, _EMBED_DIM), jnp.float32),
            pltpu.SemaphoreType.DMA,
            pltpu.SemaphoreType.DMA,
        ],
    )(xt, weight)
    return out.reshape(_NUM_FIELDS, _BATCH, _EMBED_DIM).transpose(1, 0, 2)


# speedup vs baseline: 26.5311x; 1.0068x over previous
import functools

import jax
import jax.numpy as jnp
import numpy as np
from jax import lax
from jax.experimental import pallas as pl
from jax.experimental.pallas import tpu as pltpu
from jax.experimental.pallas import tpu_sc as plsc

_EMBED_DIM = 128
_BATCH = 16384
_NUM_FIELDS = 26
_FIELD_DIM = 1000

_NC = 2
_NS = 16
_NW = _NC * _NS
_TOTAL = _BATCH * _NUM_FIELDS
_BATCH_W = _BATCH // _NW
_BLK = 128
_BPF = _BATCH_W // _BLK
_NBLK = _NUM_FIELDS * _BPF


def _body(x_hbm, w_hbm, out_hbm, idx_v, rows0, rows1, rows2, rows3, rows4,
          rows5, gsem, wsem):
    c = lax.axis_index("c")
    s = lax.axis_index("s")
    wid = s * _NC + c
    b0 = wid * _BATCH_W
    rows = (rows0, rows1, rows2, rows3, rows4, rows5)

    pltpu.sync_copy(x_hbm.at[:, pl.ds(wid * _BPF, _BPF), :], idx_v)

    def compute_idx(k, carry):
        for f in range(_NUM_FIELDS):
            for j in range(_BPF):
                sl = pl.ds(k * 16, 16)
                idx_v[f, j, sl] = idx_v[f, j, sl] + f * _FIELD_DIM
        return carry

    lax.fori_loop(0, _BLK // 16, compute_idx, 0)

    def gather(f, j, buf):
        pltpu.async_copy(w_hbm.at[idx_v.at[f, j]], buf, gsem)

    def out_at(f, j):
        return out_hbm.at[pl.ds(f * _BATCH + b0 + j * _BLK, _BLK)]

    def fld(i):
        return lax.rem(i + wid, _NUM_FIELDS)

    def emit(g_f, g_j, g_b, drain_pred, issue):
        pltpu.make_async_copy(
            w_hbm.at[idx_v.at[g_f, g_j]], rows[g_b], gsem).wait()
        pltpu.async_copy(rows[g_b], out_at(g_f, g_j), wsem)
        drain = pltpu.make_async_copy(rows[g_b], out_at(g_f, g_j), wsem)
        if drain_pred is True:
            drain.wait()
        else:

            @pl.when(drain_pred)
            def _():
                drain.wait()

        if issue is not None:
            gather(*issue)

    for g in range(3):
        gather(fld(0), g, rows[g])

    def step(k, carry):
        for u in range(12):
            f = fld(3 * k + u // 4)
            f3 = fld(3 * k + (u + 3) // 4)
            emit(f, u % 4, u % 6,
                 True if u >= 3 else (k >= 1),
                 (f3, (u + 3) % 4, rows[(u + 3) % 6]))
        return carry

    lax.fori_loop(0, 8, step, 0)

    for g in range(96, _NBLK):
        f = fld(g // 4)
        issue = None
        if g + 3 < _NBLK:
            issue = (fld((g + 3) // 4), (g + 3) % 4, rows[(g + 3) % 6])
        emit(f, g % 4, g % 6, True, issue)

    for g in range(_NBLK - 3, _NBLK):
        pltpu.make_async_copy(
            rows[g % 6], out_at(fld(g // 4), g % 4), wsem).wait()


@jax.jit
def kernel(x, weight):
    xt = x.T.reshape(_NUM_FIELDS, _BATCH // _BLK, _BLK)
    mesh = plsc.VectorSubcoreMesh(core_axis_name="c", subcore_axis_name="s")
    out = pl.kernel(
        _body,
        out_type=jax.ShapeDtypeStruct((_TOTAL, _EMBED_DIM), jnp.float32),
        mesh=mesh,
        scratch_types=[
            pltpu.VMEM((_NUM_FIELDS, _BPF, _BLK), jnp.int32),
            pltpu.VMEM((_BLK, _EMBED_DIM), jnp.float32),
            pltpu.VMEM((_BLK, _EMBED_DIM), jnp.float32),
            pltpu.VMEM((_BLK, _EMBED_DIM), jnp.float32),
            pltpu.VMEM((_BLK, _EMBED_DIM), jnp.float32),
            pltpu.VMEM((_BLK, _EMBED_DIM), jnp.float32),
            pltpu.VMEM((_BLK, _EMBED_DIM), jnp.float32),
            pltpu.SemaphoreType.DMA,
            pltpu.SemaphoreType.DMA,
        ],
    )(xt, weight)
    return out.reshape(_NUM_FIELDS, _BATCH, _EMBED_DIM).transpose(1, 0, 2)


# speedup vs baseline: 42.5951x; 1.6055x over previous
import functools

import jax
import jax.numpy as jnp
import numpy as np
from jax import lax
from jax.experimental import pallas as pl
from jax.experimental.pallas import tpu as pltpu
from jax.experimental.pallas import tpu_sc as plsc

_EMBED_DIM = 128
_BATCH = 16384
_NUM_FIELDS = 26
_FIELD_DIM = 1000

_NC = 2
_NS = 16
_NW = _NC * _NS
_TOTAL = _BATCH * _NUM_FIELDS
_BATCH_W = _BATCH // _NW
_BLK = 64
_BPF = _BATCH_W // _BLK


def _body(x_hbm, w_hbm, out_hbm, xt_v, spA, spB, rows0, rows1, rows2, rows3,
          rows4, rows5, rows6, rows7, gsem, wsem, csem):
    c = lax.axis_index("c")
    s = lax.axis_index("s")
    wid = s * _NC + c
    b0 = wid * _BATCH_W
    rows = (rows0, rows1, rows2, rows3, rows4, rows5, rows6, rows7)
    sps = (spA, spB)

    pltpu.sync_copy(x_hbm.at[:, pl.ds(b0, _BATCH_W)], xt_v)

    def fld(i):
        return lax.rem(i + c * (_NUM_FIELDS // 2), _NUM_FIELDS)

    def idx_at(f, j):
        return xt_v.at[f, pl.ds(j * _BLK, _BLK)]

    def gather(sp, f, j, buf):
        pltpu.async_copy(sp.at[idx_at(f, j)], buf, gsem)

    def out_at(f, j):
        return out_hbm.at[pl.ds(f * _BATCH + b0 + j * _BLK, _BLK)]

    def prefetch(i, sp):
        pltpu.async_copy(
            w_hbm.at[pl.ds(fld(i) * _FIELD_DIM, _FIELD_DIM)], sp, csem)

    @pl.when(s == 0)
    def _():
        prefetch(0, sps[0])

    def field_body(i, parity, k):
        sp = sps[parity]
        f = fld(i)

        @pl.when(s == 0)
        def _():
            pltpu.make_async_copy(
                w_hbm.at[pl.ds(0, _FIELD_DIM)], sp, csem).wait()

        plsc.subcore_barrier()

        fprev = fld(i - 1)
        for j in range(_BPF):
            drain = pltpu.make_async_copy(rows[j], out_at(fprev, j), wsem)
            if parity == 0:

                @pl.when(k >= 1)
                def _():
                    drain.wait()

            else:
                drain.wait()
            gather(sp, f, j, rows[j])

        nxt = sps[(parity + 1) % 2]
        if parity == 0:

            @pl.when(s == 0)
            def _():
                prefetch(i + 1, nxt)

        else:

            @pl.when(jnp.logical_and(s == 0, k < _NUM_FIELDS // 2 - 1))
            def _():
                prefetch(i + 1, nxt)

        for j in range(_BPF):
            pltpu.make_async_copy(sp.at[idx_at(f, j)], rows[j], gsem).wait()
            pltpu.async_copy(rows[j], out_at(f, j), wsem)

    def step(k, carry):
        field_body(2 * k, 0, k)
        field_body(2 * k + 1, 1, k)
        return carry

    lax.fori_loop(0, _NUM_FIELDS // 2, step, 0)

    lastf = fld(_NUM_FIELDS - 1)
    for j in range(_BPF):
        pltpu.make_async_copy(rows[j], out_at(lastf, j), wsem).wait()


@jax.jit
def kernel(x, weight):
    xt = x.T
    mesh = plsc.VectorSubcoreMesh(core_axis_name="c", subcore_axis_name="s")
    out = pl.kernel(
        _body,
        out_type=jax.ShapeDtypeStruct((_TOTAL, _EMBED_DIM), jnp.float32),
        mesh=mesh,
        scratch_types=[
            pltpu.VMEM((_NUM_FIELDS, _BATCH_W), jnp.int32),
            pltpu.VMEM_SHARED((_FIELD_DIM, _EMBED_DIM), jnp.float32),
            pltpu.VMEM_SHARED((_FIELD_DIM, _EMBED_DIM), jnp.float32),
            pltpu.VMEM((_BLK, _EMBED_DIM), jnp.float32),
            pltpu.VMEM((_BLK, _EMBED_DIM), jnp.float32),
            pltpu.VMEM((_BLK, _EMBED_DIM), jnp.float32),
            pltpu.VMEM((_BLK, _EMBED_DIM), jnp.float32),
            pltpu.VMEM((_BLK, _EMBED_DIM), jnp.float32),
            pltpu.VMEM((_BLK, _EMBED_DIM), jnp.float32),
            pltpu.VMEM((_BLK, _EMBED_DIM), jnp.float32),
            pltpu.VMEM((_BLK, _EMBED_DIM), jnp.float32),
            pltpu.SemaphoreType.DMA,
            pltpu.SemaphoreType.DMA,
            pltpu.SemaphoreType.DMA,
        ],
    )(xt, weight)
    return out.reshape(_NUM_FIELDS, _BATCH, _EMBED_DIM).transpose(1, 0, 2)
